# Initial kernel scaffold; baseline (speedup 1.0000x reference)
import functools

import jax
import jax.numpy as jnp
from jax import lax
from jax.experimental import pallas as pl
from jax.experimental.pallas import tpu as pltpu
from jax.experimental.pallas import tpu_sc as plsc

NC = 2
NS = 16
NW = NC * NS
CH = 128
NBUF = 2


def _make_gather(n_chunks: int, d: int, dtype):
    assert n_chunks % (NW * NBUF) == 0
    cpw = n_chunks // NW
    ng = cpw // NBUF

    mesh = plsc.VectorSubcoreMesh(
        core_axis_name="c", subcore_axis_name="s", num_cores=NC,
        num_subcores=NS)

    @functools.partial(
        pl.kernel,
        out_type=jax.ShapeDtypeStruct((n_chunks, CH, d), dtype),
        mesh=mesh,
        scratch_types=(
            [pltpu.VMEM((cpw, CH), jnp.int32)]
            + [pltpu.VMEM((CH, d), dtype) for _ in range(NBUF)]
            + [pltpu.SemaphoreType.DMA for _ in range(NBUF)]
        ),
    )
    def gather_kernel(table_hbm, idx_hbm, out_hbm, idx_v, *rest):
        rows = rest[:NBUF]
        sems = rest[NBUF:]
        wid = lax.axis_index("s") * NC + lax.axis_index("c")
        chunk0 = wid * cpw
        pltpu.sync_copy(idx_hbm.at[pl.ds(chunk0, cpw)], idx_v)

        def start_gather(ch, b):
            pltpu.async_copy(table_hbm.at[idx_v.at[ch]], rows[b], sems[b])

        def wait_gather(ch, b):
            pltpu.make_async_copy(
                table_hbm.at[idx_v.at[ch]], rows[b], sems[b]).wait()

        for b in range(NBUF):
            start_gather(b, b)

        @pl.loop(0, ng)
        def _(g):
            for b in range(NBUF):
                ch = g * NBUF + b
                wait_gather(ch, b)
                pltpu.sync_copy(rows[b], out_hbm.at[chunk0 + ch])

                @pl.when(g < ng - 1)
                def _():
                    start_gather(ch + NBUF, b)

    return gather_kernel


def kernel(input_ids, word_embeddings):
    bsz, seq = input_ids.shape
    _, d = word_embeddings.shape
    n = bsz * seq
    idx = input_ids.reshape(n // CH, CH).astype(jnp.int32)
    out = _make_gather(n // CH, d, word_embeddings.dtype)(
        word_embeddings, idx)
    return out.reshape(bsz, seq, d)


# speedup vs baseline: 3.3211x; 3.3211x over previous
import functools

import jax
import jax.numpy as jnp
from jax import lax
from jax.experimental import pallas as pl
from jax.experimental.pallas import tpu as pltpu
from jax.experimental.pallas import tpu_sc as plsc

NC = 2
NS = 16
NW = NC * NS
CH = 128
NBUF = 2


def _make_gather(n_chunks: int, d: int, dtype):
    assert n_chunks % (NW * NBUF) == 0
    cpw = n_chunks // NW
    ng = cpw // NBUF

    mesh = plsc.VectorSubcoreMesh(
        core_axis_name="c", subcore_axis_name="s", num_cores=NC,
        num_subcores=NS)

    @functools.partial(
        pl.kernel,
        out_type=jax.ShapeDtypeStruct((n_chunks, CH, d), dtype),
        mesh=mesh,
        scratch_types=(
            [pltpu.VMEM((cpw, CH), jnp.int32)]
            + [pltpu.VMEM((CH, d), dtype) for _ in range(NBUF)]
            + [pltpu.SemaphoreType.DMA for _ in range(NBUF)]
        ),
    )
    def gather_kernel(table_hbm, idx_hbm, out_hbm, idx_v, *rest):
        rows = rest[:NBUF]
        sems = rest[NBUF:]
        wid = lax.axis_index("s") * NC + lax.axis_index("c")
        chunk0 = wid * cpw
        pltpu.sync_copy(idx_hbm.at[wid], idx_v)

        def start_gather(ch, b):
            pltpu.async_copy(table_hbm.at[idx_v.at[ch]], rows[b], sems[b])

        def wait_gather(ch, b):
            pltpu.make_async_copy(
                table_hbm.at[idx_v.at[ch]], rows[b], sems[b]).wait()

        for b in range(NBUF):
            start_gather(b, b)

        @pl.loop(0, ng)
        def _(g):
            for b in range(NBUF):
                ch = g * NBUF + b
                wait_gather(ch, b)
                pltpu.sync_copy(rows[b], out_hbm.at[chunk0 + ch])

                @pl.when(g < ng - 1)
                def _():
                    start_gather(ch + NBUF, b)

    return gather_kernel


def kernel(input_ids, word_embeddings):
    bsz, seq = input_ids.shape
    _, d = word_embeddings.shape
    n = bsz * seq
    idx = input_ids.reshape(NW, n // (NW * CH), CH).astype(jnp.int32)
    out = _make_gather(n // CH, d, word_embeddings.dtype)(
        word_embeddings, idx)
    return out.reshape(bsz, seq, d)
